# baseline (speedup 1.0000x reference)
import functools

import jax
import jax.numpy as jnp
from jax import lax
from jax.experimental import pallas as pl
from jax.experimental.pallas import tpu as pltpu
from jax.experimental.pallas import tpu_sc as plsc

_TABLE_SIZE = 262144
_LANES = 16
_NUM_WORKERS = 32
_CHUNK = 128


def _gather_kernel(n_chunks, idx_hbm, table_hbm, out_hbm, idx_v, rows_v, gsem):
    cid = lax.axis_index("c")
    sid = lax.axis_index("s")
    wid = sid * 2 + cid
    per_w = n_chunks * _CHUNK
    base = wid * per_w

    pltpu.sync_copy(idx_hbm.at[wid], idx_v)

    offs = lax.rem(lax.iota(jnp.int32, _LANES), jnp.int32(8)) * jnp.int32(_TABLE_SIZE)

    def add_offsets(i, carry):
        r = i // (_CHUNK // _LANES)
        c = lax.rem(i, _CHUNK // _LANES)
        sl = pl.ds(c * _LANES, _LANES)
        idx_v[r, sl] = idx_v[r, sl] + offs
        return carry

    lax.fori_loop(0, per_w // _LANES, add_offsets, 0)

    def chunk_body(j, carry):
        buf = lax.rem(j, 2)
        pltpu.async_copy(table_hbm.at[idx_v.at[j]], rows_v.at[buf], gsem).wait()
        pltpu.sync_copy(rows_v.at[buf], out_hbm.at[pl.ds(base + j * _CHUNK, _CHUNK)])
        return carry

    lax.fori_loop(0, n_chunks, chunk_body, 0)


def kernel(indices, table):
    B, S, K = indices.shape
    V, D = table.shape
    N = B * S * K
    per_w = N // _NUM_WORKERS
    n_chunks = per_w // _CHUNK

    idx = indices.reshape(_NUM_WORKERS, n_chunks, _CHUNK)

    mesh = plsc.VectorSubcoreMesh(core_axis_name="c", subcore_axis_name="s")
    run = functools.partial(
        pl.kernel,
        mesh=mesh,
        out_type=jax.ShapeDtypeStruct((N, D), jnp.float32),
        scratch_types=[
            pltpu.VMEM((n_chunks, _CHUNK), jnp.int32),
            pltpu.VMEM((2, _CHUNK, D), jnp.float32),
            pltpu.SemaphoreType.DMA,
        ],
        compiler_params=pltpu.CompilerParams(use_tc_tiling_on_sc=False),
    )(functools.partial(_gather_kernel, n_chunks))

    out = run(idx, table)
    return out.reshape(B, S, K, D)


# speedup vs baseline: 1.0091x; 1.0091x over previous
import functools

import jax
import jax.numpy as jnp
from jax import lax
from jax.experimental import pallas as pl
from jax.experimental.pallas import tpu as pltpu
from jax.experimental.pallas import tpu_sc as plsc

_TABLE_SIZE = 262144
_LANES = 16
_NUM_WORKERS = 32
_CHUNK = 128


_NBUF = 15


def _gather_kernel(n_chunks, idx_hbm, table_hbm, out_hbm, idx_v, rows_v, *sems):
    cid = lax.axis_index("c")
    sid = lax.axis_index("s")
    wid = sid * 2 + cid
    per_w = n_chunks * _CHUNK
    base = wid * per_w

    pltpu.sync_copy(idx_hbm.at[wid], idx_v)

    offs = lax.rem(lax.iota(jnp.int32, _LANES), jnp.int32(8)) * jnp.int32(_TABLE_SIZE)

    def add_offsets(i, carry):
        r = i // (_CHUNK // _LANES)
        c = lax.rem(i, _CHUNK // _LANES)
        sl = pl.ds(c * _LANES, _LANES)
        idx_v[r, sl] = idx_v[r, sl] + offs
        return carry

    lax.fori_loop(0, per_w // _LANES, add_offsets, 0)

    def gather(j):
        return pltpu.async_copy(
            table_hbm.at[idx_v.at[j]], rows_v.at[j % _NBUF], sems[j % _NBUF]
        )

    def writeback(j):
        return pltpu.async_copy(
            rows_v.at[j % _NBUF],
            out_hbm.at[pl.ds(base + j * _CHUNK, _CHUNK)],
            sems[j % _NBUF],
        )

    gathers = {}
    for j in range(min(_NBUF, n_chunks)):
        gathers[j] = gather(j)
    writebacks = {}
    for j in range(n_chunks):
        gathers[j].wait()
        writebacks[j] = writeback(j)
        nxt = j + _NBUF
        if nxt < n_chunks:
            writebacks[j].wait()
            gathers[nxt] = gather(nxt)
    for j in range(max(0, n_chunks - _NBUF), n_chunks):
        if j in writebacks:
            writebacks[j].wait()


def kernel(indices, table):
    B, S, K = indices.shape
    V, D = table.shape
    N = B * S * K
    per_w = N // _NUM_WORKERS
    n_chunks = per_w // _CHUNK

    idx = indices.reshape(_NUM_WORKERS, n_chunks, _CHUNK)

    mesh = plsc.VectorSubcoreMesh(core_axis_name="c", subcore_axis_name="s")
    run = functools.partial(
        pl.kernel,
        mesh=mesh,
        out_type=jax.ShapeDtypeStruct((N, D), jnp.float32),
        scratch_types=[
            pltpu.VMEM((n_chunks, _CHUNK), jnp.int32),
            pltpu.VMEM((_NBUF, _CHUNK, D), jnp.float32),
        ]
        + [pltpu.SemaphoreType.DMA] * _NBUF,
        compiler_params=pltpu.CompilerParams(use_tc_tiling_on_sc=False),
    )(functools.partial(_gather_kernel, n_chunks))

    out = run(idx, table)
    return out.reshape(B, S, K, D)


# speedup vs baseline: 6.4541x; 6.3960x over previous
import functools

import jax
import jax.numpy as jnp
from jax import lax
from jax.experimental import pallas as pl
from jax.experimental.pallas import tpu as pltpu
from jax.experimental.pallas import tpu_sc as plsc

_LANES = 16
_NSLOT = 16


def _gather_kernel(x_hbm, i_hbm, o_hbm, stage_v, cb_v, ixb_v, obuf_v, *sems):
    gsems = sems[:_NSLOT]
    wsems = sems[_NSLOT:]
    cid = lax.axis_index("c")
    sid = lax.axis_index("s")
    w = sid * 2 + cid
    b = w // 8
    k = lax.rem(w, 8)

    pltpu.sync_copy(i_hbm.at[b], stage_v)

    kbase = k * 2097152

    def cb_body(tc, carry):
        for cblk in range(8):
            sl = pl.ds(cblk * _LANES, _LANES)
            v = stage_v[tc, k, sl]
            cb_v[tc, sl] = ((v >> 7) << 10) + (v & 127) + kbase
        return carry

    lax.fori_loop(0, 16, cb_body, 0)

    def drain_gather(slot):
        pltpu.make_async_copy(x_hbm.at[pl.ds(0, 128)], obuf_v.at[0], gsems[slot]).wait()

    def drain_write(half_idx):
        pltpu.make_async_copy(
            o_hbm.at[0, 0, 0], obuf_v.at[pl.ds(0, 128)], wsems[half_idx]
        ).wait()

    for th in range(8):
        half = th % 2
        obase = half * 128
        hbase = th * 16777216

        if th >= 2:
            drain_write(half)

        def wave_body(wv, c2, hbase=hbase, obase=obase):
            for slot in range(16):
                hh = slot % 8

                @pl.when(wv > 0)
                def _(slot=slot):
                    drain_gather(slot)

                ts = 2 * wv + slot // 8
                row = wv * 16 + slot
                for cblk in range(8):
                    sl = pl.ds(cblk * _LANES, _LANES)
                    ixb_v[slot, sl] = cb_v[ts, sl] + (hbase + hh * 128)
                pltpu.async_copy(
                    x_hbm.at[ixb_v.at[slot]],
                    obuf_v.at[obase + row],
                    gsems[slot],
                )
            return c2

        lax.fori_loop(0, 8, wave_body, 0)
        for slot in range(16):
            drain_gather(slot)
        pltpu.async_copy(
            obuf_v.at[pl.ds(obase, 128)], o_hbm.at[b, k, th], wsems[half]
        )
    drain_write(0)
    drain_write(1)


def kernel(indices, table):
    B, S, K = indices.shape
    V, D = table.shape

    x = table.T.reshape(8, 8, 16384, 128).transpose(0, 2, 1, 3).reshape(134217728)
    idx = indices.reshape(4, 16, 128, 8).transpose(0, 1, 3, 2)

    mesh = plsc.VectorSubcoreMesh(core_axis_name="c", subcore_axis_name="s")
    run = functools.partial(
        pl.kernel,
        mesh=mesh,
        out_type=jax.ShapeDtypeStruct((4, 8, 8, 128, 128), jnp.float32),
        scratch_types=[
            pltpu.VMEM((16, 8, 128), jnp.int32),
            pltpu.VMEM((16, 128), jnp.int32),
            pltpu.VMEM((_NSLOT, 128), jnp.int32),
            pltpu.VMEM((256, 128), jnp.float32),
        ]
        + [pltpu.SemaphoreType.DMA] * (_NSLOT + 2),
    )(_gather_kernel)

    out6 = run(x, idx).reshape(4, 8, 8, 16, 8, 128)
    return out6.transpose(0, 3, 5, 1, 2, 4).reshape(B, S, K, D)


# speedup vs baseline: 6.6337x; 1.0278x over previous
import functools

import jax
import jax.numpy as jnp
from jax import lax
from jax.experimental import pallas as pl
from jax.experimental.pallas import tpu as pltpu
from jax.experimental.pallas import tpu_sc as plsc

_LANES = 16
_NSLOT = 16


def _gather_kernel(x_hbm, i_hbm, o_hbm, stage_v, cb_v, ixb_v, obuf_v, *sems):
    gsems = sems[:_NSLOT]
    wsems = sems[_NSLOT:]
    cid = lax.axis_index("c")
    sid = lax.axis_index("s")
    w = sid * 2 + cid
    b = w // 8
    k = lax.rem(w, 8)

    pltpu.sync_copy(i_hbm.at[b, :, k], stage_v)

    kbase = k * 2097152

    def cb_body(tc, carry):
        for cblk in range(8):
            sl = pl.ds(cblk * _LANES, _LANES)
            v = stage_v[tc, sl]
            cb_v[tc, sl] = ((v >> 7) << 10) + (v & 127) + kbase
        return carry

    lax.fori_loop(0, 16, cb_body, 0)

    def drain_gather(slot):
        pltpu.make_async_copy(x_hbm.at[pl.ds(0, 128)], obuf_v.at[0], gsems[slot]).wait()

    def drain_write(half_idx):
        pltpu.make_async_copy(
            o_hbm.at[0, 0, 0], obuf_v.at[pl.ds(0, 128)], wsems[half_idx]
        ).wait()

    for th in range(8):
        half = th % 2
        obase = half * 128
        hbase = th * 16777216

        if th >= 2:
            drain_write(half)

        def wave_body(wv, c2, th=th, hbase=hbase, obase=obase):
            for slot in range(16):
                hh = slot % 8

                if th == 0:

                    @pl.when(wv > 0)
                    def _(slot=slot):
                        drain_gather(slot)

                else:
                    drain_gather(slot)

                ts = 2 * wv + slot // 8
                row = wv * 16 + slot
                for cblk in range(8):
                    sl = pl.ds(cblk * _LANES, _LANES)
                    ixb_v[slot, sl] = cb_v[ts, sl] + (hbase + hh * 128)
                pltpu.async_copy(
                    x_hbm.at[ixb_v.at[slot]],
                    obuf_v.at[obase + row],
                    gsems[slot],
                )
            if th > 0:
                @pl.when(wv == 0)
                def _(th=th, obase=obase):
                    pltpu.async_copy(
                        obuf_v.at[pl.ds(128 - obase, 128)],
                        o_hbm.at[b, k, th - 1],
                        wsems[1 - (th % 2)],
                    )

            return c2

        lax.fori_loop(0, 8, wave_body, 0)
    for slot in range(16):
        drain_gather(slot)
    pltpu.async_copy(obuf_v.at[pl.ds(128, 128)], o_hbm.at[b, k, 7], wsems[1])
    drain_write(0)
    drain_write(1)


def kernel(indices, table):
    B, S, K = indices.shape
    V, D = table.shape

    x = table.T.reshape(8, 8, 16384, 128).transpose(0, 2, 1, 3).reshape(134217728)
    idx = indices.reshape(4, 16, 128, 8).transpose(0, 1, 3, 2)

    mesh = plsc.VectorSubcoreMesh(core_axis_name="c", subcore_axis_name="s")
    run = functools.partial(
        pl.kernel,
        mesh=mesh,
        out_type=jax.ShapeDtypeStruct((4, 8, 8, 128, 128), jnp.float32),
        scratch_types=[
            pltpu.VMEM((16, 128), jnp.int32),
            pltpu.VMEM((16, 128), jnp.int32),
            pltpu.VMEM((_NSLOT, 128), jnp.int32),
            pltpu.VMEM((256, 128), jnp.float32),
        ]
        + [pltpu.SemaphoreType.DMA] * (_NSLOT + 2),
    )(_gather_kernel)

    out6 = run(x, idx).reshape(4, 8, 8, 16, 8, 128)
    return out6.transpose(0, 3, 5, 1, 2, 4).reshape(B, S, K, D)
